# baseline (speedup 1.0000x reference)
import functools

import numpy as np
import jax
import jax.numpy as jnp
from jax import lax
from jax.experimental import pallas as pl
from jax.experimental.pallas import tpu as pltpu
from jax.experimental.pallas import tpu_sc as plsc

VOCAB = 100000
D = 1024
DH = D // 2
B = 4
S = 4096

NC = 2
NS = 16
NW = NC * NS
POS_PER_W = S // NW
CHUNK = 8
NCHUNK = POS_PER_W // CHUNK
LANES = 16
NROWBUF = 3


def _pe_table_half() -> np.ndarray:
    pos = np.arange(S, dtype=np.float32)[:, None]
    a = np.arange(0, D, 2)
    ang = (1.0 / np.power(10000.0, a.astype(np.float64) / D)).astype(np.float32)[None, :]
    pa = (pos * ang).astype(np.float32)
    pa[0::2] = np.sin(pa[0::2])
    pa[1::2] = np.cos(pa[1::2])
    return pa


_PE_HALF = _pe_table_half()


def _pe_bf16_shuffled() -> np.ndarray:
    import ml_dtypes
    h = _PE_HALF.reshape(S, DH // 32, 2, 16)
    h = np.ascontiguousarray(np.transpose(h, (0, 1, 3, 2))).reshape(S * DH)
    return h.astype(ml_dtypes.bfloat16).view(np.int32)


def _emb_pe_body(x_hbm, pe_hbm, table_hbm, out_hbm,
                 idx_all, rows_v, pe_v, gsem, psem, osem):
    wid = lax.axis_index("s") * NC + lax.axis_index("c")
    base = pl.multiple_of(wid * POS_PER_W, POS_PER_W)

    for b in range(B):
        pltpu.sync_copy(x_hbm.at[b, pl.ds(base, POS_PER_W)], idx_all.at[b])

    il = lax.iota(jnp.int32, LANES)
    lane_half = il >> 1
    lane_hi = lane_half + (LANES // 2)
    _gd = lax.GatherDimensionNumbers(
        offset_dims=(), collapsed_slice_dims=(0,), start_index_map=(0,))

    def _lane_dup(vec, idx):
        return lax.gather(vec, idx[:, None], _gd, slice_sizes=(1,),
                          mode=lax.GatherScatterMode.PROMISE_IN_BOUNDS)

    pend_g = {}
    pend_o = {}

    def issue(c):
        r = c % NROWBUF
        q = c % 2
        if c - NROWBUF in pend_o:
            for dd in pend_o.pop(c - NROWBUF):
                dd.wait()
        descs = []
        for b in range(B):
            d = pltpu.make_async_copy(
                table_hbm.at[idx_all.at[b, pl.ds(c * CHUNK, CHUNK)]],
                rows_v.at[r, pl.ds(b * CHUNK, CHUNK)],
                gsem.at[r])
            d.start()
            descs.append(d)
        nw = CHUNK * DH // 2
        dpe = pltpu.make_async_copy(
            pe_hbm.at[pl.ds((base + c * CHUNK) * (DH // 2), nw)],
            pe_v.at[pl.ds(q * nw, nw)], psem.at[q])
        dpe.start()
        descs.append(dpe)
        pend_g[c] = descs

    def compute(c):
        r = c % NROWBUF
        q = c % 2

        JU = 1
        UNROLL = 2

        def j_body(j2, carry):
            j0 = pl.multiple_of(j2 * JU, JU)

            def v_body(v, carry2):
                h0 = pl.multiple_of(v * UNROLL * LANES, UNROLL * LANES)
                for ju in range(JU):
                    j = j0 + ju
                    poff = pl.multiple_of(
                        (q * CHUNK * DH + j * DH + h0) // 2, LANES)
                    w = pe_v[pl.ds(poff, LANES)]
                    pha = lax.bitcast_convert_type(w << 16, jnp.float32)
                    phc = lax.bitcast_convert_type(
                        w & jnp.int32(-65536), jnp.float32)
                    for u, ph in ((0, pha), (1, phc)):
                        hcol = h0 + u * LANES
                        plo = _lane_dup(ph, lane_half)
                        phi = _lane_dup(ph, lane_hi)
                        col = hcol * 2
                        for b in range(B):
                            rr = b * CHUNK + j
                            plsc.addupdate(
                                rows_v.at[r, rr, pl.ds(col, LANES)], plo)
                            plsc.addupdate(
                                rows_v.at[r, rr, pl.ds(col + LANES, LANES)],
                                phi)
                return carry2

            lax.fori_loop(0, DH // (UNROLL * LANES), v_body, 0)
            return carry

        lax.fori_loop(0, CHUNK // JU, j_body, 0)

    issue(0)
    for c in range(NCHUNK):
        if c + 1 < NCHUNK:
            issue(c + 1)
        for d in pend_g.pop(c):
            d.wait()
        compute(c)
        r = c % NROWBUF
        outs = []
        for b in range(B):
            d = pltpu.make_async_copy(
                rows_v.at[r, pl.ds(b * CHUNK, CHUNK)],
                out_hbm.at[pl.ds(b * S + base + c * CHUNK, CHUNK)],
                osem.at[r])
            d.start()
            outs.append(d)
        pend_o[c] = outs
    for c in sorted(pend_o):
        for d in pend_o[c]:
            d.wait()


@functools.cache
def _build_emb_pe():
    mesh = plsc.VectorSubcoreMesh(core_axis_name="c", subcore_axis_name="s")

    @functools.partial(
        pl.kernel,
        mesh=mesh,
        out_type=jax.ShapeDtypeStruct((B * S, D), jnp.float32),
        scratch_types=[
            pltpu.VMEM((B, POS_PER_W), jnp.int32),
            pltpu.VMEM((NROWBUF, B * CHUNK, D), jnp.float32),
            pltpu.VMEM((CHUNK * DH,), jnp.int32),
            pltpu.SemaphoreType.DMA((NROWBUF,)),
            pltpu.SemaphoreType.DMA((2,)),
            pltpu.SemaphoreType.DMA((NROWBUF,)),
        ],
    )
    def _emb_pe(x_hbm, pe_hbm, table_hbm, out_hbm,
                idx_all, rows_v, pe_v, gsem, psem, osem):
        _emb_pe_body(x_hbm, pe_hbm, table_hbm, out_hbm,
                     idx_all, rows_v, pe_v, gsem, psem, osem)

    return _emb_pe


@functools.cache
def _pe_device():
    return jax.device_put(_pe_bf16_shuffled())


def kernel(x, table):
    xi = x.astype(jnp.int32)
    out = _build_emb_pe()(xi, _pe_device(), table)
    return out.reshape(B, S, D)
